# Initial kernel scaffold; baseline (speedup 1.0000x reference)
import jax
import jax.numpy as jnp
from jax.experimental import pallas as pl


def kernel(x, edge_index, edge_attr, batch, W_enc, b_enc, eW1_0, eb1_0, eW2_0, eb2_0, Wr_0, br_0, gamma_0, beta_0, eW1_1, eb1_1, eW2_1, eb2_1, Wr_1, br_1, gamma_1, beta_1, Wh1, bh1, Wh2, bh2):
    raise NotImplementedError("write your pallas kernel here")


# speedup vs baseline: 1.0075x; 1.0075x over previous
import functools

import jax
import jax.numpy as jnp
from jax import lax
from jax.experimental import pallas as pl
from jax.experimental.pallas import tpu as pltpu
from jax.experimental.pallas import tpu_sc as plsc

N = 10000
E = 160000
H = 32

NC = 2
NS = 16
NW = NC * NS
EPW = E // NW
CHUNK = 125
CPW = EPW // CHUNK
GB = 8
GROUPS = CPW // GB

EB = 2000
G = 64


def _sc_mesh():
    return plsc.VectorSubcoreMesh(
        core_axis_name="c", subcore_axis_name="s", num_cores=NC, num_subcores=NS
    )


@functools.cache
def _get_sc_gather():
    return functools.partial(
        pl.kernel,
        out_type=jax.ShapeDtypeStruct((E, H), jnp.float32),
        mesh=_sc_mesh(),
        compiler_params=pltpu.CompilerParams(use_tc_tiling_on_sc=False),
        scratch_types=[
            pltpu.VMEM((CPW, CHUNK), jnp.int32),
            pltpu.VMEM((GB * CHUNK, H), jnp.float32),
            pltpu.SemaphoreType.DMA,
        ],
    )(_sc_gather_body)


def _sc_gather_body(h_hbm, srcT_hbm, hs_hbm, idx_v, buf_v, sem):
    c = lax.axis_index("c")
    s = lax.axis_index("s")
    wid = s * NC + c
    base = wid * EPW
    pltpu.sync_copy(srcT_hbm.at[wid], idx_v)

    def group_body(g, carry):
        descs = []
        for j in range(GB):
            d = pltpu.async_copy(
                h_hbm.at[idx_v.at[g * GB + j]],
                buf_v.at[pl.ds(j * CHUNK, CHUNK)],
                sem,
            )
            descs.append(d)
        for d in descs:
            d.wait()
        pltpu.sync_copy(buf_v, hs_hbm.at[pl.ds(base + g * (GB * CHUNK), GB * CHUNK)])
        return carry

    lax.fori_loop(0, GROUPS, group_body, 0)


def _make_sc_scatter(with_deg):
    out_type = [jax.ShapeDtypeStruct((NC, N, H), jnp.float32)]
    scratch = [
        pltpu.VMEM((CPW, CHUNK), jnp.int32),
        pltpu.VMEM((GB * CHUNK, H), jnp.float32),
        pltpu.VMEM_SHARED((N, H), jnp.float32),
        pltpu.SemaphoreType.DMA,
    ]
    if with_deg:
        out_type.append(jax.ShapeDtypeStruct((NC, N, H), jnp.float32))
        scratch.insert(2, pltpu.VMEM((CHUNK, H), jnp.float32))
        scratch.insert(3, pltpu.VMEM_SHARED((N, H), jnp.float32))

    RPS = N // NS

    def body(msg_hbm, dstT_hbm, z_hbm, ones_hbm, *refs):
        if with_deg:
            (agg_hbm, deg_hbm, idx_v, buf_v, ones_v, deg_sh, acc_sh, sem) = refs
        else:
            (agg_hbm, idx_v, buf_v, acc_sh, sem) = refs
        c = lax.axis_index("c")
        s = lax.axis_index("s")
        wid = s * NC + c
        base = wid * EPW

        rows = pl.ds(s * RPS, RPS)
        pltpu.sync_copy(z_hbm.at[rows], acc_sh.at[rows])
        if with_deg:
            pltpu.sync_copy(z_hbm.at[rows], deg_sh.at[rows])
            pltpu.sync_copy(ones_hbm, ones_v)
        pltpu.sync_copy(dstT_hbm.at[wid], idx_v)
        plsc.subcore_barrier()

        def group_body(g, carry):
            pltpu.async_copy(
                msg_hbm.at[pl.ds(base + g * (GB * CHUNK), GB * CHUNK)], buf_v, sem
            ).wait()
            for j in range(GB):
                pltpu.sync_copy(
                    buf_v.at[pl.ds(j * CHUNK, CHUNK)],
                    acc_sh.at[idx_v.at[g * GB + j]],
                    add=True,
                )
                if with_deg:
                    pltpu.sync_copy(ones_v, deg_sh.at[idx_v.at[g * GB + j]], add=True)
            return carry

        lax.fori_loop(0, GROUPS, group_body, 0)
        plsc.subcore_barrier()

        pltpu.sync_copy(acc_sh.at[rows], agg_hbm.at[c].at[rows])
        if with_deg:
            pltpu.sync_copy(deg_sh.at[rows], deg_hbm.at[c].at[rows])

    return pl.kernel(
        body,
        out_type=tuple(out_type) if with_deg else out_type[0],
        mesh=_sc_mesh(),
        compiler_params=pltpu.CompilerParams(use_tc_tiling_on_sc=False),
        scratch_types=scratch,
    )


@functools.cache
def _get_sc_scatter(with_deg):
    return _make_sc_scatter(with_deg)


def _enc_body(x_ref, w_ref, b_ref, out_ref):
    out_ref[...] = (
        jnp.dot(x_ref[...], w_ref[...], preferred_element_type=jnp.float32)
        + b_ref[...]
    )


def _msg_body(ea_ref, hs_ref, w1_ref, b1_ref, w2_ref, b2_ref, out_ref):
    eh = jnp.maximum(
        jnp.dot(ea_ref[...], w1_ref[...], preferred_element_type=jnp.float32)
        + b1_ref[...],
        0.0,
    )
    we = (
        jnp.dot(eh, w2_ref[...], preferred_element_type=jnp.float32) + b2_ref[...]
    )
    hs = hs_ref[...]
    acc = hs[:, 0:1] * we[:, 0:H]
    for i in range(1, H):
        acc = acc + hs[:, i : i + 1] * we[:, i * H : (i + 1) * H]
    out_ref[...] = acc


def _node_body(aggp_ref, degp_ref, h_ref, wr_ref, br_ref, g_ref, b_ref, out_ref):
    agg = aggp_ref[0:N, :] + aggp_ref[N : 2 * N, :]
    deg = jnp.maximum(degp_ref[0:N, 0:1] + degp_ref[N : 2 * N, 0:1], 1.0)
    h = h_ref[...]
    hn = agg / deg + jnp.dot(h, wr_ref[...], preferred_element_type=jnp.float32) + br_ref[...]
    mu = jnp.mean(hn, axis=0, keepdims=True)
    var = jnp.mean((hn - mu) ** 2, axis=0, keepdims=True)
    hn = g_ref[...] * (hn - mu) / jnp.sqrt(var + 1e-5) + b_ref[...]
    out_ref[...] = h + jnp.maximum(hn, 0.0)


def _head_body(h_ref, batch_ref, w1_ref, b1_ref, w2_ref, b2_ref, out_ref):
    gid = lax.broadcasted_iota(jnp.int32, (G, N), 0)
    onehot = (batch_ref[...] == gid).astype(jnp.float32)
    ssum = jnp.dot(onehot, h_ref[...], preferred_element_type=jnp.float32)
    cnt = jnp.maximum(jnp.sum(onehot, axis=1, keepdims=True), 1.0)
    gm = ssum / cnt
    hidden = jnp.maximum(
        jnp.dot(gm, w1_ref[...], preferred_element_type=jnp.float32) + b1_ref[...], 0.0
    )
    out_ref[...] = (
        jnp.dot(hidden, w2_ref[...], preferred_element_type=jnp.float32) + b2_ref[...]
    )


def _tc_enc(x, w, b):
    return pl.pallas_call(
        _enc_body,
        out_shape=jax.ShapeDtypeStruct((N, H), jnp.float32),
    )(x, w, b.reshape(1, H))


def _tc_msg(ea, hs, w1, b1, w2, b2):
    grid = (E // EB,)
    return pl.pallas_call(
        _msg_body,
        grid=grid,
        in_specs=[
            pl.BlockSpec((EB, 16), lambda i: (i, 0)),
            pl.BlockSpec((EB, H), lambda i: (i, 0)),
            pl.BlockSpec((16, 128), lambda i: (0, 0)),
            pl.BlockSpec((1, 128), lambda i: (0, 0)),
            pl.BlockSpec((128, H * H), lambda i: (0, 0)),
            pl.BlockSpec((1, H * H), lambda i: (0, 0)),
        ],
        out_specs=pl.BlockSpec((EB, H), lambda i: (i, 0)),
        out_shape=jax.ShapeDtypeStruct((E, H), jnp.float32),
    )(ea, hs, w1, b1.reshape(1, 128), w2, b2.reshape(1, H * H))


def _tc_node(aggp, degp, h, wr, br, gamma, beta):
    return pl.pallas_call(
        _node_body,
        out_shape=jax.ShapeDtypeStruct((N, H), jnp.float32),
    )(
        aggp.reshape(2 * N, H),
        degp.reshape(2 * N, H),
        h,
        wr,
        br.reshape(1, H),
        gamma.reshape(1, H),
        beta.reshape(1, H),
    )


def _tc_head(h, batch, wh1, bh1, wh2, bh2):
    out = pl.pallas_call(
        _head_body,
        out_shape=jax.ShapeDtypeStruct((G, 1), jnp.float32),
    )(
        h,
        batch.reshape(1, N),
        wh1,
        bh1.reshape(1, H),
        wh2,
        bh2.reshape(1, 1),
    )
    return out.reshape(G)


def kernel(x, edge_index, edge_attr, batch, W_enc, b_enc, eW1_0, eb1_0, eW2_0, eb2_0, Wr_0, br_0, gamma_0, beta_0, eW1_1, eb1_1, eW2_1, eb2_1, Wr_1, br_1, gamma_1, beta_1, Wh1, bh1, Wh2, bh2):
    srcT = edge_index[0].reshape(NW, CPW, CHUNK)
    dstT = edge_index[1].reshape(NW, CPW, CHUNK)
    zeros = jnp.zeros((N, H), jnp.float32)
    ones = jnp.ones((CHUNK, H), jnp.float32)

    h = _tc_enc(x, W_enc, b_enc)

    hs = _get_sc_gather()(h, srcT)
    msg = _tc_msg(edge_attr, hs, eW1_0, eb1_0, eW2_0, eb2_0)
    aggp, degp = _get_sc_scatter(True)(msg, dstT, zeros, ones)
    h = _tc_node(aggp, degp, h, Wr_0, br_0, gamma_0, beta_0)

    hs = _get_sc_gather()(h, srcT)
    msg = _tc_msg(edge_attr, hs, eW1_1, eb1_1, eW2_1, eb2_1)
    aggp = _get_sc_scatter(False)(msg, dstT, zeros, ones)
    h = _tc_node(aggp, degp, h, Wr_1, br_1, gamma_1, beta_1)

    return _tc_head(h, batch, Wh1, bh1, Wh2, bh2)


# speedup vs baseline: 2.7916x; 2.7709x over previous
import functools

import jax
import jax.numpy as jnp
from jax import lax
from jax.experimental import pallas as pl
from jax.experimental.pallas import tpu as pltpu
from jax.experimental.pallas import tpu_sc as plsc

N = 10000
E = 160000
H = 32

NC = 2
NS = 16
NW = NC * NS
EPW = E // NW
CHUNK = 125
CPW = EPW // CHUNK
GB = 8
GROUPS = CPW // GB

EB = 2000
G = 64


def _sc_mesh():
    return plsc.VectorSubcoreMesh(
        core_axis_name="c", subcore_axis_name="s", num_cores=NC, num_subcores=NS
    )


@functools.cache
def _get_sc_gather():
    return functools.partial(
        pl.kernel,
        out_type=jax.ShapeDtypeStruct((E, H), jnp.float32),
        mesh=_sc_mesh(),
        compiler_params=pltpu.CompilerParams(use_tc_tiling_on_sc=False),
        scratch_types=[
            pltpu.VMEM((CPW, CHUNK), jnp.int32),
            pltpu.VMEM((GB * CHUNK, H), jnp.float32),
            pltpu.SemaphoreType.DMA,
        ],
    )(_sc_gather_body)


def _sc_gather_body(h_hbm, srcT_hbm, hs_hbm, idx_v, buf_v, sem):
    c = lax.axis_index("c")
    s = lax.axis_index("s")
    wid = s * NC + c
    base = wid * EPW
    pltpu.sync_copy(srcT_hbm.at[wid], idx_v)

    def group_body(g, carry):
        descs = []
        for j in range(GB):
            d = pltpu.async_copy(
                h_hbm.at[idx_v.at[g * GB + j]],
                buf_v.at[pl.ds(j * CHUNK, CHUNK)],
                sem,
            )
            descs.append(d)
        for d in descs:
            d.wait()
        pltpu.sync_copy(buf_v, hs_hbm.at[pl.ds(base + g * (GB * CHUNK), GB * CHUNK)])
        return carry

    lax.fori_loop(0, GROUPS, group_body, 0)


def _make_sc_scatter(with_deg):
    out_type = [jax.ShapeDtypeStruct((NC, N, H), jnp.float32)]
    scratch = [
        pltpu.VMEM((CPW, CHUNK), jnp.int32),
        pltpu.VMEM((GB * CHUNK, H), jnp.float32),
        pltpu.VMEM_SHARED((N, H), jnp.float32),
        pltpu.SemaphoreType.DMA,
    ]
    if with_deg:
        out_type.append(jax.ShapeDtypeStruct((NC, N, H), jnp.float32))
        scratch.insert(2, pltpu.VMEM((CHUNK, H), jnp.float32))
        scratch.insert(3, pltpu.VMEM_SHARED((N, H), jnp.float32))

    RPS = N // NS

    def body(msg_hbm, dstT_hbm, z_hbm, ones_hbm, *refs):
        if with_deg:
            (agg_hbm, deg_hbm, idx_v, buf_v, ones_v, deg_sh, acc_sh, sem) = refs
        else:
            (agg_hbm, idx_v, buf_v, acc_sh, sem) = refs
        c = lax.axis_index("c")
        s = lax.axis_index("s")
        wid = s * NC + c
        base = wid * EPW

        rows = pl.ds(s * RPS, RPS)
        pltpu.sync_copy(z_hbm.at[rows], acc_sh.at[rows])
        if with_deg:
            pltpu.sync_copy(z_hbm.at[rows], deg_sh.at[rows])
            pltpu.sync_copy(ones_hbm, ones_v)
        pltpu.sync_copy(dstT_hbm.at[wid], idx_v)
        plsc.subcore_barrier()

        def group_body(g, carry):
            pltpu.async_copy(
                msg_hbm.at[pl.ds(base + g * (GB * CHUNK), GB * CHUNK)], buf_v, sem
            ).wait()
            for j in range(GB):
                pltpu.sync_copy(
                    buf_v.at[pl.ds(j * CHUNK, CHUNK)],
                    acc_sh.at[idx_v.at[g * GB + j]],
                    add=True,
                )
                if with_deg:
                    pltpu.sync_copy(ones_v, deg_sh.at[idx_v.at[g * GB + j]], add=True)
            return carry

        lax.fori_loop(0, GROUPS, group_body, 0)
        plsc.subcore_barrier()

        pltpu.sync_copy(acc_sh.at[rows], agg_hbm.at[c].at[rows])
        if with_deg:
            pltpu.sync_copy(deg_sh.at[rows], deg_hbm.at[c].at[rows])

    return pl.kernel(
        body,
        out_type=tuple(out_type) if with_deg else out_type[0],
        mesh=_sc_mesh(),
        compiler_params=pltpu.CompilerParams(use_tc_tiling_on_sc=False),
        scratch_types=scratch,
    )


@functools.cache
def _get_sc_scatter(with_deg):
    return _make_sc_scatter(with_deg)


def _enc_body(x_ref, w_ref, b_ref, out_ref):
    out_ref[...] = (
        jnp.dot(x_ref[...], w_ref[...], preferred_element_type=jnp.float32)
        + b_ref[...]
    )


def _msg_body(ea_ref, hs_ref, w1_ref, b1_ref, w2p_ref, b2p_ref, tile_ref, sum_ref, out_ref):
    eh = jnp.maximum(
        jnp.dot(ea_ref[...], w1_ref[...], preferred_element_type=jnp.float32)
        + b1_ref[...],
        0.0,
    )
    we = jnp.dot(eh, w2p_ref[...], preferred_element_type=jnp.float32) + b2p_ref[...]
    hstile = jnp.dot(hs_ref[...], tile_ref[...], preferred_element_type=jnp.float32)
    out_ref[...] = jnp.dot(
        we * hstile, sum_ref[...], preferred_element_type=jnp.float32
    )


def _node_body(aggp_ref, degp_ref, h_ref, wr_ref, br_ref, g_ref, b_ref, out_ref):
    agg = aggp_ref[0:N, :] + aggp_ref[N : 2 * N, :]
    deg = jnp.maximum(degp_ref[0:N, 0:1] + degp_ref[N : 2 * N, 0:1], 1.0)
    h = h_ref[...]
    hn = agg / deg + jnp.dot(h, wr_ref[...], preferred_element_type=jnp.float32) + br_ref[...]
    mu = jnp.mean(hn, axis=0, keepdims=True)
    var = jnp.mean((hn - mu) ** 2, axis=0, keepdims=True)
    hn = g_ref[...] * (hn - mu) / jnp.sqrt(var + 1e-5) + b_ref[...]
    out_ref[...] = h + jnp.maximum(hn, 0.0)


def _head_body(h_ref, batch_ref, w1_ref, b1_ref, w2_ref, b2_ref, out_ref):
    gid = lax.broadcasted_iota(jnp.int32, (G, N), 0)
    onehot = (batch_ref[...] == gid).astype(jnp.float32)
    ssum = jnp.dot(onehot, h_ref[...], preferred_element_type=jnp.float32)
    cnt = jnp.maximum(jnp.sum(onehot, axis=1, keepdims=True), 1.0)
    gm = ssum / cnt
    hidden = jnp.maximum(
        jnp.dot(gm, w1_ref[...], preferred_element_type=jnp.float32) + b1_ref[...], 0.0
    )
    out_ref[...] = (
        jnp.dot(hidden, w2_ref[...], preferred_element_type=jnp.float32) + b2_ref[...]
    )


def _tc_enc(x, w, b):
    return pl.pallas_call(
        _enc_body,
        out_shape=jax.ShapeDtypeStruct((N, H), jnp.float32),
    )(x, w, b.reshape(1, H))


def _tc_msg(ea, hs, w1, b1, w2, b2):
    perm = (jnp.arange(H * H) % H) * H + jnp.arange(H * H) // H
    w2p = w2[:, perm]
    b2p = b2[perm].reshape(1, H * H)
    tile = jnp.tile(jnp.eye(H, dtype=jnp.float32), (1, H))
    gsum = jnp.repeat(jnp.eye(H, dtype=jnp.float32), H, axis=0)
    grid = (E // EB,)
    return pl.pallas_call(
        _msg_body,
        grid=grid,
        in_specs=[
            pl.BlockSpec((EB, 16), lambda i: (i, 0)),
            pl.BlockSpec((EB, H), lambda i: (i, 0)),
            pl.BlockSpec((16, 128), lambda i: (0, 0)),
            pl.BlockSpec((1, 128), lambda i: (0, 0)),
            pl.BlockSpec((128, H * H), lambda i: (0, 0)),
            pl.BlockSpec((1, H * H), lambda i: (0, 0)),
            pl.BlockSpec((H, H * H), lambda i: (0, 0)),
            pl.BlockSpec((H * H, H), lambda i: (0, 0)),
        ],
        out_specs=pl.BlockSpec((EB, H), lambda i: (i, 0)),
        out_shape=jax.ShapeDtypeStruct((E, H), jnp.float32),
    )(ea, hs, w1, b1.reshape(1, 128), w2p, b2p, tile, gsum)


def _tc_node(aggp, degp, h, wr, br, gamma, beta):
    return pl.pallas_call(
        _node_body,
        out_shape=jax.ShapeDtypeStruct((N, H), jnp.float32),
    )(
        aggp.reshape(2 * N, H),
        degp.reshape(2 * N, H),
        h,
        wr,
        br.reshape(1, H),
        gamma.reshape(1, H),
        beta.reshape(1, H),
    )


def _tc_head(h, batch, wh1, bh1, wh2, bh2):
    out = pl.pallas_call(
        _head_body,
        out_shape=jax.ShapeDtypeStruct((G, 1), jnp.float32),
    )(
        h,
        batch.reshape(1, N),
        wh1,
        bh1.reshape(1, H),
        wh2,
        bh2.reshape(1, 1),
    )
    return out.reshape(G)


def kernel(x, edge_index, edge_attr, batch, W_enc, b_enc, eW1_0, eb1_0, eW2_0, eb2_0, Wr_0, br_0, gamma_0, beta_0, eW1_1, eb1_1, eW2_1, eb2_1, Wr_1, br_1, gamma_1, beta_1, Wh1, bh1, Wh2, bh2):
    srcT = edge_index[0].reshape(NW, CPW, CHUNK)
    dstT = edge_index[1].reshape(NW, CPW, CHUNK)
    zeros = jnp.zeros((N, H), jnp.float32)
    ones = jnp.ones((CHUNK, H), jnp.float32)

    h = _tc_enc(x, W_enc, b_enc)

    hs = _get_sc_gather()(h, srcT)
    msg = _tc_msg(edge_attr, hs, eW1_0, eb1_0, eW2_0, eb2_0)
    aggp, degp = _get_sc_scatter(True)(msg, dstT, zeros, ones)
    h = _tc_node(aggp, degp, h, Wr_0, br_0, gamma_0, beta_0)

    hs = _get_sc_gather()(h, srcT)
    msg = _tc_msg(edge_attr, hs, eW1_1, eb1_1, eW2_1, eb2_1)
    aggp = _get_sc_scatter(False)(msg, dstT, zeros, ones)
    h = _tc_node(aggp, degp, h, Wr_1, br_1, gamma_1, beta_1)

    return _tc_head(h, batch, Wh1, bh1, Wh2, bh2)


# speedup vs baseline: 2.9510x; 1.0571x over previous
import functools

import jax
import jax.numpy as jnp
from jax import lax
from jax.experimental import pallas as pl
from jax.experimental.pallas import tpu as pltpu
from jax.experimental.pallas import tpu_sc as plsc

N = 10000
E = 160000
H = 32

NC = 2
NS = 16
NW = NC * NS
EPW = E // NW
CHUNK = 125
CPW = EPW // CHUNK
GB = 8
GROUPS = CPW // GB

EB = 2000
G = 64


def _sc_mesh():
    return plsc.VectorSubcoreMesh(
        core_axis_name="c", subcore_axis_name="s", num_cores=NC, num_subcores=NS
    )


@functools.cache
def _get_sc_gather():
    return functools.partial(
        pl.kernel,
        out_type=jax.ShapeDtypeStruct((E, H), jnp.float32),
        mesh=_sc_mesh(),
        compiler_params=pltpu.CompilerParams(use_tc_tiling_on_sc=False),
        scratch_types=[
            pltpu.VMEM((CPW, CHUNK), jnp.int32),
            pltpu.VMEM((GB * CHUNK, H), jnp.float32),
            pltpu.SemaphoreType.DMA,
        ],
    )(_sc_gather_body)


def _sc_gather_body(h_hbm, srcT_hbm, hs_hbm, idx_v, buf_v, sem):
    c = lax.axis_index("c")
    s = lax.axis_index("s")
    wid = s * NC + c
    base = wid * EPW
    pltpu.sync_copy(srcT_hbm.at[wid], idx_v)

    def group_body(g, carry):
        descs = []
        for j in range(GB):
            d = pltpu.async_copy(
                h_hbm.at[idx_v.at[g * GB + j]],
                buf_v.at[pl.ds(j * CHUNK, CHUNK)],
                sem,
            )
            descs.append(d)
        for d in descs:
            d.wait()
        pltpu.sync_copy(buf_v, hs_hbm.at[pl.ds(base + g * (GB * CHUNK), GB * CHUNK)])
        return carry

    lax.fori_loop(0, GROUPS, group_body, 0)


def _make_sc_scatter(with_deg):
    out_type = [jax.ShapeDtypeStruct((NC, N, H), jnp.float32)]
    scratch = [
        pltpu.VMEM((CPW, CHUNK), jnp.int32),
        pltpu.VMEM((GB * CHUNK, H), jnp.float32),
        pltpu.VMEM_SHARED((N, H), jnp.float32),
        pltpu.SemaphoreType.DMA,
    ]
    if with_deg:
        out_type.append(jax.ShapeDtypeStruct((NC, N, H), jnp.float32))
        scratch.insert(2, pltpu.VMEM((CHUNK, H), jnp.float32))
        scratch.insert(3, pltpu.VMEM_SHARED((N, H), jnp.float32))

    RPS = N // NS

    def body(msg_hbm, dstT_hbm, z_hbm, ones_hbm, *refs):
        if with_deg:
            (agg_hbm, deg_hbm, idx_v, buf_v, ones_v, deg_sh, acc_sh, sem) = refs
        else:
            (agg_hbm, idx_v, buf_v, acc_sh, sem) = refs
        c = lax.axis_index("c")
        s = lax.axis_index("s")
        wid = s * NC + c
        base = wid * EPW

        rows = pl.ds(s * RPS, RPS)
        pltpu.sync_copy(z_hbm.at[rows], acc_sh.at[rows])
        if with_deg:
            pltpu.sync_copy(z_hbm.at[rows], deg_sh.at[rows])
            pltpu.sync_copy(ones_hbm, ones_v)
        pltpu.sync_copy(dstT_hbm.at[wid], idx_v)
        plsc.subcore_barrier()

        def group_body(g, carry):
            pltpu.async_copy(
                msg_hbm.at[pl.ds(base + g * (GB * CHUNK), GB * CHUNK)], buf_v, sem
            ).wait()
            for j in range(GB):
                pltpu.sync_copy(
                    buf_v.at[pl.ds(j * CHUNK, CHUNK)],
                    acc_sh.at[idx_v.at[g * GB + j]],
                    add=True,
                )
                if with_deg:
                    pltpu.sync_copy(ones_v, deg_sh.at[idx_v.at[g * GB + j]], add=True)
            return carry

        lax.fori_loop(0, GROUPS, group_body, 0)
        plsc.subcore_barrier()

        pltpu.sync_copy(acc_sh.at[rows], agg_hbm.at[c].at[rows])
        if with_deg:
            pltpu.sync_copy(deg_sh.at[rows], deg_hbm.at[c].at[rows])

    return pl.kernel(
        body,
        out_type=tuple(out_type) if with_deg else out_type[0],
        mesh=_sc_mesh(),
        compiler_params=pltpu.CompilerParams(use_tc_tiling_on_sc=False),
        scratch_types=scratch,
    )


@functools.cache
def _get_sc_scatter(with_deg):
    return _make_sc_scatter(with_deg)


def _enc_body(x_ref, w_ref, b_ref, out_ref):
    out_ref[...] = (
        jnp.dot(x_ref[...], w_ref[...], preferred_element_type=jnp.float32)
        + b_ref[...]
    )


def _msg_body(ea_ref, hs_ref, w1_ref, b1_ref, w2p_ref, b2p_ref, tile_ref, sum_ref, out_ref):
    eh = jnp.maximum(
        jnp.dot(ea_ref[...], w1_ref[...], preferred_element_type=jnp.float32)
        + b1_ref[...],
        0.0,
    )
    we = (
        jnp.dot(
            eh.astype(jnp.bfloat16),
            w2p_ref[...].astype(jnp.bfloat16),
            preferred_element_type=jnp.float32,
        )
        + b2p_ref[...]
    )
    hstile = jnp.dot(
        hs_ref[...].astype(jnp.bfloat16),
        tile_ref[...].astype(jnp.bfloat16),
        preferred_element_type=jnp.float32,
    )
    out_ref[...] = jnp.dot(
        we * hstile, sum_ref[...], preferred_element_type=jnp.float32
    )


def _node_body(aggp_ref, degp_ref, h_ref, wr_ref, br_ref, g_ref, b_ref, out_ref):
    agg = aggp_ref[0:N, :] + aggp_ref[N : 2 * N, :]
    deg = jnp.maximum(degp_ref[0:N, 0:1] + degp_ref[N : 2 * N, 0:1], 1.0)
    h = h_ref[...]
    hn = agg / deg + jnp.dot(h, wr_ref[...], preferred_element_type=jnp.float32) + br_ref[...]
    mu = jnp.mean(hn, axis=0, keepdims=True)
    var = jnp.mean((hn - mu) ** 2, axis=0, keepdims=True)
    hn = g_ref[...] * (hn - mu) / jnp.sqrt(var + 1e-5) + b_ref[...]
    out_ref[...] = h + jnp.maximum(hn, 0.0)


def _head_body(h_ref, batch_ref, w1_ref, b1_ref, w2_ref, b2_ref, out_ref):
    gid = lax.broadcasted_iota(jnp.int32, (G, N), 0)
    onehot = (batch_ref[...] == gid).astype(jnp.float32)
    ssum = jnp.dot(onehot, h_ref[...], preferred_element_type=jnp.float32)
    cnt = jnp.maximum(jnp.sum(onehot, axis=1, keepdims=True), 1.0)
    gm = ssum / cnt
    hidden = jnp.maximum(
        jnp.dot(gm, w1_ref[...], preferred_element_type=jnp.float32) + b1_ref[...], 0.0
    )
    out_ref[...] = (
        jnp.dot(hidden, w2_ref[...], preferred_element_type=jnp.float32) + b2_ref[...]
    )


def _tc_enc(x, w, b):
    return pl.pallas_call(
        _enc_body,
        out_shape=jax.ShapeDtypeStruct((N, H), jnp.float32),
    )(x, w, b.reshape(1, H))


def _tc_msg(ea, hs, w1, b1, w2, b2):
    perm = (jnp.arange(H * H) % H) * H + jnp.arange(H * H) // H
    w2p = w2[:, perm]
    b2p = b2[perm].reshape(1, H * H)
    tile = jnp.tile(jnp.eye(H, dtype=jnp.float32), (1, H))
    gsum = jnp.repeat(jnp.eye(H, dtype=jnp.float32), H, axis=0)
    grid = (E // EB,)
    return pl.pallas_call(
        _msg_body,
        grid=grid,
        in_specs=[
            pl.BlockSpec((EB, 16), lambda i: (i, 0)),
            pl.BlockSpec((EB, H), lambda i: (i, 0)),
            pl.BlockSpec((16, 128), lambda i: (0, 0)),
            pl.BlockSpec((1, 128), lambda i: (0, 0)),
            pl.BlockSpec((128, H * H), lambda i: (0, 0)),
            pl.BlockSpec((1, H * H), lambda i: (0, 0)),
            pl.BlockSpec((H, H * H), lambda i: (0, 0)),
            pl.BlockSpec((H * H, H), lambda i: (0, 0)),
        ],
        out_specs=pl.BlockSpec((EB, H), lambda i: (i, 0)),
        out_shape=jax.ShapeDtypeStruct((E, H), jnp.float32),
    )(ea, hs, w1, b1.reshape(1, 128), w2p, b2p, tile, gsum)


def _tc_node(aggp, degp, h, wr, br, gamma, beta):
    return pl.pallas_call(
        _node_body,
        out_shape=jax.ShapeDtypeStruct((N, H), jnp.float32),
    )(
        aggp.reshape(2 * N, H),
        degp.reshape(2 * N, H),
        h,
        wr,
        br.reshape(1, H),
        gamma.reshape(1, H),
        beta.reshape(1, H),
    )


def _tc_head(h, batch, wh1, bh1, wh2, bh2):
    out = pl.pallas_call(
        _head_body,
        out_shape=jax.ShapeDtypeStruct((G, 1), jnp.float32),
    )(
        h,
        batch.reshape(1, N),
        wh1,
        bh1.reshape(1, H),
        wh2,
        bh2.reshape(1, 1),
    )
    return out.reshape(G)


def kernel(x, edge_index, edge_attr, batch, W_enc, b_enc, eW1_0, eb1_0, eW2_0, eb2_0, Wr_0, br_0, gamma_0, beta_0, eW1_1, eb1_1, eW2_1, eb2_1, Wr_1, br_1, gamma_1, beta_1, Wh1, bh1, Wh2, bh2):
    srcT = edge_index[0].reshape(NW, CPW, CHUNK)
    dstT = edge_index[1].reshape(NW, CPW, CHUNK)
    zeros = jnp.zeros((N, H), jnp.float32)
    ones = jnp.ones((CHUNK, H), jnp.float32)

    h = _tc_enc(x, W_enc, b_enc)

    hs = _get_sc_gather()(h, srcT)
    msg = _tc_msg(edge_attr, hs, eW1_0, eb1_0, eW2_0, eb2_0)
    aggp, degp = _get_sc_scatter(True)(msg, dstT, zeros, ones)
    h = _tc_node(aggp, degp, h, Wr_0, br_0, gamma_0, beta_0)

    hs = _get_sc_gather()(h, srcT)
    msg = _tc_msg(edge_attr, hs, eW1_1, eb1_1, eW2_1, eb2_1)
    aggp = _get_sc_scatter(False)(msg, dstT, zeros, ones)
    h = _tc_node(aggp, degp, h, Wr_1, br_1, gamma_1, beta_1)

    return _tc_head(h, batch, Wh1, bh1, Wh2, bh2)


# speedup vs baseline: 3.4204x; 1.1591x over previous
import functools

import jax
import jax.numpy as jnp
from jax import lax
from jax.experimental import pallas as pl
from jax.experimental.pallas import tpu as pltpu
from jax.experimental.pallas import tpu_sc as plsc

N = 10000
E = 160000
H = 32

NC = 2
NS = 16
NW = NC * NS
EPW = E // NW
CHUNK = 125
CPW = EPW // CHUNK
GB = 8
GROUPS = CPW // GB

EB = 2000
G = 64


def _sc_mesh():
    return plsc.VectorSubcoreMesh(
        core_axis_name="c", subcore_axis_name="s", num_cores=NC, num_subcores=NS
    )


@functools.cache
def _get_sc_gather():
    return functools.partial(
        pl.kernel,
        out_type=jax.ShapeDtypeStruct((E, H), jnp.float32),
        mesh=_sc_mesh(),
        compiler_params=pltpu.CompilerParams(use_tc_tiling_on_sc=False),
        scratch_types=[
            pltpu.VMEM((CPW, CHUNK), jnp.int32),
            pltpu.VMEM((GB * CHUNK, H), jnp.float32),
            pltpu.SemaphoreType.DMA,
        ],
    )(_sc_gather_body)


def _sc_gather_body(h_hbm, srcT_hbm, hs_hbm, idx_v, buf_v, sem):
    c = lax.axis_index("c")
    s = lax.axis_index("s")
    wid = s * NC + c
    base = wid * EPW
    pltpu.sync_copy(srcT_hbm.at[wid], idx_v)

    def group_body(g, carry):
        descs = []
        for j in range(GB):
            d = pltpu.async_copy(
                h_hbm.at[idx_v.at[g * GB + j]],
                buf_v.at[pl.ds(j * CHUNK, CHUNK)],
                sem,
            )
            descs.append(d)
        for d in descs:
            d.wait()
        pltpu.sync_copy(buf_v, hs_hbm.at[pl.ds(base + g * (GB * CHUNK), GB * CHUNK)])
        return carry

    lax.fori_loop(0, GROUPS, group_body, 0)


def _make_sc_scatter(with_deg):
    out_type = [jax.ShapeDtypeStruct((NC, N, H), jnp.float32)]
    scratch = [
        pltpu.VMEM((CPW, CHUNK), jnp.int32),
        pltpu.VMEM((GB * CHUNK, H), jnp.float32),
        pltpu.VMEM_SHARED((N, H), jnp.float32),
        pltpu.SemaphoreType.DMA,
    ]
    if with_deg:
        out_type.append(jax.ShapeDtypeStruct((NC, N, H), jnp.float32))
        scratch.insert(2, pltpu.VMEM((CHUNK, H), jnp.float32))
        scratch.insert(3, pltpu.VMEM_SHARED((N, H), jnp.float32))

    RPS = N // NS

    def body(msg_hbm, dstT_hbm, z_hbm, ones_hbm, *refs):
        if with_deg:
            (agg_hbm, deg_hbm, idx_v, buf_v, ones_v, deg_sh, acc_sh, sem) = refs
        else:
            (agg_hbm, idx_v, buf_v, acc_sh, sem) = refs
        c = lax.axis_index("c")
        s = lax.axis_index("s")
        wid = s * NC + c
        base = wid * EPW

        rows = pl.ds(s * RPS, RPS)
        pltpu.sync_copy(z_hbm.at[rows], acc_sh.at[rows])
        if with_deg:
            pltpu.sync_copy(z_hbm.at[rows], deg_sh.at[rows])
            pltpu.sync_copy(ones_hbm, ones_v)
        pltpu.sync_copy(dstT_hbm.at[wid], idx_v)
        plsc.subcore_barrier()

        def group_body(g, carry):
            pltpu.async_copy(
                msg_hbm.at[pl.ds(base + g * (GB * CHUNK), GB * CHUNK)], buf_v, sem
            ).wait()
            for j in range(GB):
                pltpu.sync_copy(
                    buf_v.at[pl.ds(j * CHUNK, CHUNK)],
                    acc_sh.at[idx_v.at[g * GB + j]],
                    add=True,
                )
                if with_deg:
                    pltpu.sync_copy(ones_v, deg_sh.at[idx_v.at[g * GB + j]], add=True)
            return carry

        lax.fori_loop(0, GROUPS, group_body, 0)
        plsc.subcore_barrier()

        pltpu.sync_copy(acc_sh.at[rows], agg_hbm.at[c].at[rows])
        if with_deg:
            pltpu.sync_copy(deg_sh.at[rows], deg_hbm.at[c].at[rows])

    return pl.kernel(
        body,
        out_type=tuple(out_type) if with_deg else out_type[0],
        mesh=_sc_mesh(),
        compiler_params=pltpu.CompilerParams(use_tc_tiling_on_sc=False),
        scratch_types=scratch,
    )


@functools.cache
def _get_sc_scatter(with_deg):
    return _make_sc_scatter(with_deg)


def _enc_body(x_ref, w_ref, b_ref, out_ref):
    out_ref[...] = (
        jnp.dot(x_ref[...], w_ref[...], preferred_element_type=jnp.float32)
        + b_ref[...]
    )


def _msg_body(ea_ref, hs_ref, w1_ref, b1_ref, w2_ref, rep_ref, red_ref, b2m_ref, out_ref):
    eh = jnp.maximum(
        jnp.dot(ea_ref[...], w1_ref[...], preferred_element_type=jnp.float32)
        + b1_ref[...],
        0.0,
    )
    we = jnp.dot(
        eh.astype(jnp.bfloat16), w2_ref[...], preferred_element_type=jnp.float32
    )
    hs = hs_ref[...]
    hsr = jnp.dot(
        hs.astype(jnp.bfloat16), rep_ref[...], preferred_element_type=jnp.float32
    )
    prod = we * hsr
    part = prod[:, 0:128]
    for t in range(1, (H * H) // 128):
        part = part + prod[:, t * 128 : (t + 1) * 128]
    out_ref[...] = (
        jnp.dot(part, red_ref[...], preferred_element_type=jnp.float32)
        + jnp.dot(hs, b2m_ref[...], preferred_element_type=jnp.float32)
    )


def _node_body(aggp_ref, degp_ref, h_ref, wr_ref, br_ref, g_ref, b_ref, out_ref):
    agg = aggp_ref[0:N, :] + aggp_ref[N : 2 * N, :]
    deg = jnp.maximum(degp_ref[0:N, 0:1] + degp_ref[N : 2 * N, 0:1], 1.0)
    h = h_ref[...]
    hn = agg / deg + jnp.dot(h, wr_ref[...], preferred_element_type=jnp.float32) + br_ref[...]
    mu = jnp.mean(hn, axis=0, keepdims=True)
    var = jnp.mean((hn - mu) ** 2, axis=0, keepdims=True)
    hn = g_ref[...] * (hn - mu) / jnp.sqrt(var + 1e-5) + b_ref[...]
    out_ref[...] = h + jnp.maximum(hn, 0.0)


def _head_body(h_ref, batch_ref, w1_ref, b1_ref, w2_ref, b2_ref, out_ref):
    gid = lax.broadcasted_iota(jnp.int32, (G, N), 0)
    onehot = (batch_ref[...] == gid).astype(jnp.float32)
    ssum = jnp.dot(onehot, h_ref[...], preferred_element_type=jnp.float32)
    cnt = jnp.maximum(jnp.sum(onehot, axis=1, keepdims=True), 1.0)
    gm = ssum / cnt
    hidden = jnp.maximum(
        jnp.dot(gm, w1_ref[...], preferred_element_type=jnp.float32) + b1_ref[...], 0.0
    )
    out_ref[...] = (
        jnp.dot(hidden, w2_ref[...], preferred_element_type=jnp.float32) + b2_ref[...]
    )


def _tc_enc(x, w, b):
    return pl.pallas_call(
        _enc_body,
        out_shape=jax.ShapeDtypeStruct((N, H), jnp.float32),
    )(x, w, b.reshape(1, H))


def _tc_msg(ea, hs, w1, b1, w2, b2):
    w2b = w2.astype(jnp.bfloat16)
    rep = jnp.repeat(jnp.eye(H, dtype=jnp.bfloat16), H, axis=1)
    red = jnp.tile(jnp.eye(H, dtype=jnp.float32), (4, 1))
    b2m = b2.reshape(H, H)
    grid = (E // EB,)
    return pl.pallas_call(
        _msg_body,
        grid=grid,
        in_specs=[
            pl.BlockSpec((EB, 16), lambda i: (i, 0)),
            pl.BlockSpec((EB, H), lambda i: (i, 0)),
            pl.BlockSpec((16, 128), lambda i: (0, 0)),
            pl.BlockSpec((1, 128), lambda i: (0, 0)),
            pl.BlockSpec((128, H * H), lambda i: (0, 0)),
            pl.BlockSpec((H, H * H), lambda i: (0, 0)),
            pl.BlockSpec((128, H), lambda i: (0, 0)),
            pl.BlockSpec((H, H), lambda i: (0, 0)),
        ],
        out_specs=pl.BlockSpec((EB, H), lambda i: (i, 0)),
        out_shape=jax.ShapeDtypeStruct((E, H), jnp.float32),
    )(ea, hs, w1, b1.reshape(1, 128), w2b, rep, red, b2m)


def _tc_node(aggp, degp, h, wr, br, gamma, beta):
    return pl.pallas_call(
        _node_body,
        out_shape=jax.ShapeDtypeStruct((N, H), jnp.float32),
    )(
        aggp.reshape(2 * N, H),
        degp.reshape(2 * N, H),
        h,
        wr,
        br.reshape(1, H),
        gamma.reshape(1, H),
        beta.reshape(1, H),
    )


def _tc_head(h, batch, wh1, bh1, wh2, bh2):
    out = pl.pallas_call(
        _head_body,
        out_shape=jax.ShapeDtypeStruct((G, 1), jnp.float32),
    )(
        h,
        batch.reshape(1, N),
        wh1,
        bh1.reshape(1, H),
        wh2,
        bh2.reshape(1, 1),
    )
    return out.reshape(G)


def kernel(x, edge_index, edge_attr, batch, W_enc, b_enc, eW1_0, eb1_0, eW2_0, eb2_0, Wr_0, br_0, gamma_0, beta_0, eW1_1, eb1_1, eW2_1, eb2_1, Wr_1, br_1, gamma_1, beta_1, Wh1, bh1, Wh2, bh2):
    srcT = edge_index[0].reshape(NW, CPW, CHUNK)
    dstT = edge_index[1].reshape(NW, CPW, CHUNK)
    zeros = jnp.zeros((N, H), jnp.float32)
    ones = jnp.ones((CHUNK, H), jnp.float32)

    h = _tc_enc(x, W_enc, b_enc)

    hs = _get_sc_gather()(h, srcT)
    msg = _tc_msg(edge_attr, hs, eW1_0, eb1_0, eW2_0, eb2_0)
    aggp, degp = _get_sc_scatter(True)(msg, dstT, zeros, ones)
    h = _tc_node(aggp, degp, h, Wr_0, br_0, gamma_0, beta_0)

    hs = _get_sc_gather()(h, srcT)
    msg = _tc_msg(edge_attr, hs, eW1_1, eb1_1, eW2_1, eb2_1)
    aggp = _get_sc_scatter(False)(msg, dstT, zeros, ones)
    h = _tc_node(aggp, degp, h, Wr_1, br_1, gamma_1, beta_1)

    return _tc_head(h, batch, Wh1, bh1, Wh2, bh2)


# speedup vs baseline: 3.5124x; 1.0269x over previous
import functools

import jax
import jax.numpy as jnp
from jax import lax
from jax.experimental import pallas as pl
from jax.experimental.pallas import tpu as pltpu
from jax.experimental.pallas import tpu_sc as plsc

N = 10000
E = 160000
H = 32

NC = 2
NS = 16
NW = NC * NS
EPW = E // NW
CHUNK = 125
CPW = EPW // CHUNK
GB = 8
GROUPS = CPW // GB

EB = 4000
G = 64


def _sc_mesh():
    return plsc.VectorSubcoreMesh(
        core_axis_name="c", subcore_axis_name="s", num_cores=NC, num_subcores=NS
    )


@functools.cache
def _get_sc_gather():
    return functools.partial(
        pl.kernel,
        out_type=jax.ShapeDtypeStruct((E, H), jnp.float32),
        mesh=_sc_mesh(),
        compiler_params=pltpu.CompilerParams(use_tc_tiling_on_sc=False),
        scratch_types=[
            pltpu.VMEM((CPW, CHUNK), jnp.int32),
            pltpu.VMEM((GB * CHUNK, H), jnp.float32),
            pltpu.SemaphoreType.DMA,
        ],
    )(_sc_gather_body)


def _sc_gather_body(h_hbm, srcT_hbm, hs_hbm, idx_v, buf_v, sem):
    c = lax.axis_index("c")
    s = lax.axis_index("s")
    wid = s * NC + c
    base = wid * EPW
    pltpu.sync_copy(srcT_hbm.at[wid], idx_v)

    def group_body(g, carry):
        descs = []
        for j in range(GB):
            d = pltpu.async_copy(
                h_hbm.at[idx_v.at[g * GB + j]],
                buf_v.at[pl.ds(j * CHUNK, CHUNK)],
                sem,
            )
            descs.append(d)
        for d in descs:
            d.wait()
        pltpu.sync_copy(buf_v, hs_hbm.at[pl.ds(base + g * (GB * CHUNK), GB * CHUNK)])
        return carry

    lax.fori_loop(0, GROUPS, group_body, 0)


def _make_sc_scatter(with_deg):
    out_type = [jax.ShapeDtypeStruct((NC, N, H), jnp.float32)]
    scratch = [
        pltpu.VMEM((CPW, CHUNK), jnp.int32),
        pltpu.VMEM((GB * CHUNK, H), jnp.float32),
        pltpu.VMEM_SHARED((N, H), jnp.float32),
        pltpu.SemaphoreType.DMA,
    ]
    if with_deg:
        out_type.append(jax.ShapeDtypeStruct((NC, N, H), jnp.float32))
        scratch.insert(2, pltpu.VMEM((CHUNK, H), jnp.float32))
        scratch.insert(3, pltpu.VMEM_SHARED((N, H), jnp.float32))

    RPS = N // NS

    def body(msg_hbm, dstT_hbm, z_hbm, ones_hbm, *refs):
        if with_deg:
            (agg_hbm, deg_hbm, idx_v, buf_v, ones_v, deg_sh, acc_sh, sem) = refs
        else:
            (agg_hbm, idx_v, buf_v, acc_sh, sem) = refs
        c = lax.axis_index("c")
        s = lax.axis_index("s")
        wid = s * NC + c
        base = wid * EPW

        rows = pl.ds(s * RPS, RPS)
        pltpu.sync_copy(z_hbm.at[rows], acc_sh.at[rows])
        if with_deg:
            pltpu.sync_copy(z_hbm.at[rows], deg_sh.at[rows])
            pltpu.sync_copy(ones_hbm, ones_v)
        pltpu.sync_copy(dstT_hbm.at[wid], idx_v)
        plsc.subcore_barrier()

        def group_body(g, carry):
            pltpu.async_copy(
                msg_hbm.at[pl.ds(base + g * (GB * CHUNK), GB * CHUNK)], buf_v, sem
            ).wait()
            for j in range(GB):
                pltpu.sync_copy(
                    buf_v.at[pl.ds(j * CHUNK, CHUNK)],
                    acc_sh.at[idx_v.at[g * GB + j]],
                    add=True,
                )
                if with_deg:
                    pltpu.sync_copy(ones_v, deg_sh.at[idx_v.at[g * GB + j]], add=True)
            return carry

        lax.fori_loop(0, GROUPS, group_body, 0)
        plsc.subcore_barrier()

        pltpu.sync_copy(acc_sh.at[rows], agg_hbm.at[c].at[rows])
        if with_deg:
            pltpu.sync_copy(deg_sh.at[rows], deg_hbm.at[c].at[rows])

    return pl.kernel(
        body,
        out_type=tuple(out_type) if with_deg else out_type[0],
        mesh=_sc_mesh(),
        compiler_params=pltpu.CompilerParams(use_tc_tiling_on_sc=False),
        scratch_types=scratch,
    )


@functools.cache
def _get_sc_scatter(with_deg):
    return _make_sc_scatter(with_deg)


def _enc_body(x_ref, w_ref, b_ref, out_ref):
    out_ref[...] = (
        jnp.dot(x_ref[...], w_ref[...], preferred_element_type=jnp.float32)
        + b_ref[...]
    )


def _msg_body(ea_ref, hs_ref, w1_ref, b1_ref, w2_ref, rep_ref, red_ref, b2m_ref, out_ref):
    eh = jnp.maximum(
        jnp.dot(ea_ref[...], w1_ref[...], preferred_element_type=jnp.float32)
        + b1_ref[...],
        0.0,
    )
    we = jnp.dot(
        eh.astype(jnp.bfloat16), w2_ref[...], preferred_element_type=jnp.float32
    )
    hs = hs_ref[...]
    hsr = jnp.dot(
        hs.astype(jnp.bfloat16), rep_ref[...], preferred_element_type=jnp.float32
    )
    prod = we * hsr
    part = prod[:, 0:128]
    for t in range(1, (H * H) // 128):
        part = part + prod[:, t * 128 : (t + 1) * 128]
    out_ref[...] = (
        jnp.dot(part, red_ref[...], preferred_element_type=jnp.float32)
        + jnp.dot(hs, b2m_ref[...], preferred_element_type=jnp.float32)
    )


def _node_body(aggp_ref, degp_ref, h_ref, wr_ref, br_ref, g_ref, b_ref, out_ref):
    agg = aggp_ref[0:N, :] + aggp_ref[N : 2 * N, :]
    deg = jnp.maximum(degp_ref[0:N, 0:1] + degp_ref[N : 2 * N, 0:1], 1.0)
    h = h_ref[...]
    hn = agg / deg + jnp.dot(h, wr_ref[...], preferred_element_type=jnp.float32) + br_ref[...]
    mu = jnp.mean(hn, axis=0, keepdims=True)
    var = jnp.mean((hn - mu) ** 2, axis=0, keepdims=True)
    hn = g_ref[...] * (hn - mu) / jnp.sqrt(var + 1e-5) + b_ref[...]
    out_ref[...] = h + jnp.maximum(hn, 0.0)


def _head_body(h_ref, batch_ref, w1_ref, b1_ref, w2_ref, b2_ref, out_ref):
    gid = lax.broadcasted_iota(jnp.int32, (G, N), 0)
    onehot = (batch_ref[...] == gid).astype(jnp.float32)
    ssum = jnp.dot(onehot, h_ref[...], preferred_element_type=jnp.float32)
    cnt = jnp.maximum(jnp.sum(onehot, axis=1, keepdims=True), 1.0)
    gm = ssum / cnt
    hidden = jnp.maximum(
        jnp.dot(gm, w1_ref[...], preferred_element_type=jnp.float32) + b1_ref[...], 0.0
    )
    out_ref[...] = (
        jnp.dot(hidden, w2_ref[...], preferred_element_type=jnp.float32) + b2_ref[...]
    )


def _tc_enc(x, w, b):
    return pl.pallas_call(
        _enc_body,
        out_shape=jax.ShapeDtypeStruct((N, H), jnp.float32),
    )(x, w, b.reshape(1, H))


def _tc_msg(ea, hs, w1, b1, w2, b2):
    w2b = w2.astype(jnp.bfloat16)
    rep = jnp.repeat(jnp.eye(H, dtype=jnp.bfloat16), H, axis=1)
    red = jnp.tile(jnp.eye(H, dtype=jnp.float32), (4, 1))
    b2m = b2.reshape(H, H)
    grid = (E // EB,)
    return pl.pallas_call(
        _msg_body,
        grid=grid,
        in_specs=[
            pl.BlockSpec((EB, 16), lambda i: (i, 0)),
            pl.BlockSpec((EB, H), lambda i: (i, 0)),
            pl.BlockSpec((16, 128), lambda i: (0, 0)),
            pl.BlockSpec((1, 128), lambda i: (0, 0)),
            pl.BlockSpec((128, H * H), lambda i: (0, 0)),
            pl.BlockSpec((H, H * H), lambda i: (0, 0)),
            pl.BlockSpec((128, H), lambda i: (0, 0)),
            pl.BlockSpec((H, H), lambda i: (0, 0)),
        ],
        out_specs=pl.BlockSpec((EB, H), lambda i: (i, 0)),
        out_shape=jax.ShapeDtypeStruct((E, H), jnp.float32),
    )(ea, hs, w1, b1.reshape(1, 128), w2b, rep, red, b2m)


def _tc_node(aggp, degp, h, wr, br, gamma, beta):
    return pl.pallas_call(
        _node_body,
        out_shape=jax.ShapeDtypeStruct((N, H), jnp.float32),
    )(
        aggp.reshape(2 * N, H),
        degp.reshape(2 * N, H),
        h,
        wr,
        br.reshape(1, H),
        gamma.reshape(1, H),
        beta.reshape(1, H),
    )


def _tc_head(h, batch, wh1, bh1, wh2, bh2):
    out = pl.pallas_call(
        _head_body,
        out_shape=jax.ShapeDtypeStruct((G, 1), jnp.float32),
    )(
        h,
        batch.reshape(1, N),
        wh1,
        bh1.reshape(1, H),
        wh2,
        bh2.reshape(1, 1),
    )
    return out.reshape(G)


def kernel(x, edge_index, edge_attr, batch, W_enc, b_enc, eW1_0, eb1_0, eW2_0, eb2_0, Wr_0, br_0, gamma_0, beta_0, eW1_1, eb1_1, eW2_1, eb2_1, Wr_1, br_1, gamma_1, beta_1, Wh1, bh1, Wh2, bh2):
    srcT = edge_index[0].reshape(NW, CPW, CHUNK)
    dstT = edge_index[1].reshape(NW, CPW, CHUNK)
    zeros = jnp.zeros((N, H), jnp.float32)
    ones = jnp.ones((CHUNK, H), jnp.float32)

    h = _tc_enc(x, W_enc, b_enc)

    hs = _get_sc_gather()(h, srcT)
    msg = _tc_msg(edge_attr, hs, eW1_0, eb1_0, eW2_0, eb2_0)
    aggp, degp = _get_sc_scatter(True)(msg, dstT, zeros, ones)
    h = _tc_node(aggp, degp, h, Wr_0, br_0, gamma_0, beta_0)

    hs = _get_sc_gather()(h, srcT)
    msg = _tc_msg(edge_attr, hs, eW1_1, eb1_1, eW2_1, eb2_1)
    aggp = _get_sc_scatter(False)(msg, dstT, zeros, ones)
    h = _tc_node(aggp, degp, h, Wr_1, br_1, gamma_1, beta_1)

    return _tc_head(h, batch, Wh1, bh1, Wh2, bh2)
